# baseline (speedup 1.0000x reference)
import functools

import jax
import jax.numpy as jnp
from jax import lax
from jax.experimental import pallas as pl
from jax.experimental.pallas import tpu as pltpu
from jax.experimental.pallas import tpu_sc as plsc

_D = 128
_B = 512
_NW = 32
_BW = _B // _NW
_L = 16


def _sc_partials(table, nfall):
    mesh = plsc.VectorSubcoreMesh(core_axis_name="c", subcore_axis_name="s")
    f32 = jnp.float32
    i32 = jnp.int32

    @functools.partial(
        pl.kernel,
        mesh=mesh,
        compiler_params=pltpu.CompilerParams(
            use_tc_tiling_on_sc=False, needs_layout_passes=False),
        out_type=jax.ShapeDtypeStruct((4 * _L * _NW,), f32),
        scratch_types=(
            pltpu.VMEM((8 * _BW,), i32),
            pltpu.VMEM((2 * _BW, 2 * _D), f32),
            pltpu.VMEM((3 * _BW, 2 * _D), f32),
            pltpu.VMEM((3 * _BW, 2 * _D), f32),
            pltpu.VMEM((_BW * (_L + 1),), f32),
            pltpu.VMEM((_BW * (_L + 1),), f32),
            pltpu.VMEM((4 * _L,), f32),
            pltpu.SemaphoreType.DMA,
            pltpu.SemaphoreType.DMA,
            pltpu.SemaphoreType.DMA,
            pltpu.SemaphoreType.DMA,
        ),
    )
    def k(table_hbm, nf_hbm, out_hbm,
          xi, v1, v2, v3, accma, accmb, sall,
          sim, g1m, g2m, g3m):
        wid = lax.axis_index("s") * 2 + lax.axis_index("c")

        pltpu.async_copy(
            nf_hbm.at[pl.ds(wid * 8 * _BW, 8 * _BW)], xi, sim).wait()

        cp1 = pltpu.async_copy(table_hbm.at[xi.at[pl.ds(0, 2 * _BW)]], v1, g1m)
        cp2 = pltpu.async_copy(
            table_hbm.at[xi.at[pl.ds(2 * _BW, 3 * _BW)]], v2, g2m)
        cp3 = pltpu.async_copy(
            table_hbm.at[xi.at[pl.ds(5 * _BW, 3 * _BW)]], v3, g3m)

        lanes = lax.broadcasted_iota(i32, (_BW,), 0)

        cp1.wait()

        def body1(r, acc):
            rd = r + _BW

            def chunk1(j, a):
                lo = pl.ds(_L * j, _L)
                hi = pl.ds(_D + _L * j, _L)
                c1 = v1[r, lo]
                cr = v1[r, hi]
                d1 = v1[rd, lo]
                dr = v1[rd, hi]
                t = jnp.maximum(
                    jnp.abs(c1 - d1) + jnp.abs(cr) - jnp.abs(dr), 0.0)
                return a + t * t

            return lax.fori_loop(0, _D // _L, chunk1, acc)

        acc1 = plsc.parallel_loop(
            0, _BW, unroll=1, carry=jnp.zeros((_L,), f32))(body1)
        sall[pl.ds(0, _L)] = acc1

        cp2.wait()

        @plsc.parallel_loop(0, _BW, unroll=1)
        def body2(r):
            rd = r + _BW
            re = r + 2 * _BW

            def chunk2(j, accs):
                aa, ab = accs
                lo = pl.ds(_L * j, _L)
                hi = pl.ds(_D + _L * j, _L)
                c1 = v2[r, lo]
                c2 = jnp.abs(v2[r, hi])
                d1 = v2[rd, lo]
                d2 = jnp.abs(v2[rd, hi])
                e1 = v2[re, lo]
                e2 = jnp.abs(v2[re, hi])
                start = jnp.maximum(c1 - c2, d1 - d2)
                end = jnp.minimum(c1 + c2, d1 + d2)
                diff = start - end
                cen = (start + end) * 0.5
                t1 = jnp.maximum(
                    jnp.abs(cen - e1) + jnp.abs(diff) * 0.5 - e2, 0.0)
                t2 = jnp.maximum(diff, 0.0)
                return (aa + t1 * t1, ab + t2 * t2)

            aa, ab = lax.fori_loop(
                0, _D // _L, chunk2,
                (jnp.zeros((_L,), f32), jnp.zeros((_L,), f32)))
            accma[pl.ds(r * (_L + 1), _L)] = aa
            accmb[pl.ds(r * (_L + 1), _L)] = ab

        cp3.wait()

        def body3(r, acc):
            rd = r + _BW
            rr_ = r + 2 * _BW

            def chunk3(j, a):
                lo = pl.ds(_L * j, _L)
                hi = pl.ds(_D + _L * j, _L)
                c1 = v3[r, lo]
                cr = v3[r, hi]
                d1 = v3[rd, lo]
                dr = v3[rd, hi]
                rr = v3[rr_, lo]
                t = jnp.maximum(
                    jnp.abs(c1 + rr - d1) + jnp.abs(cr) - jnp.abs(dr), 0.0)
                return a + t * t

            return lax.fori_loop(0, _D // _L, chunk3, acc)

        acc3 = plsc.parallel_loop(
            0, _BW, unroll=1, carry=jnp.zeros((_L,), f32))(body3)
        sall[pl.ds(3 * _L, _L)] = acc3

        sa = jnp.zeros((_L,), f32)
        sb = jnp.zeros((_L,), f32)
        lanes17 = lanes * (_L + 1)
        for c in range(_L):
            sa = sa + plsc.load_gather(accma, [lanes17 + c])
            sb = sb + plsc.load_gather(accmb, [lanes17 + c])
        sall[pl.ds(_L, _L)] = sa
        sall[pl.ds(2 * _L, _L)] = sb

        pltpu.sync_copy(sall, out_hbm.at[pl.ds(wid * 4 * _L, 4 * _L)])

    return k(table, nfall)


def _reduce_body(p_ref, o_ref):
    p = p_ref[...]

    def blk(k):
        return jnp.sum(p[:, k * _L:(k + 1) * _L]) \
            + jnp.sum(p[:, 64 + k * _L:64 + (k + 1) * _L])

    def blk_sqrt(k):
        return jnp.sum(jnp.sqrt(p[:, k * _L:(k + 1) * _L])) \
            + jnp.sum(jnp.sqrt(p[:, 64 + k * _L:64 + (k + 1) * _L]))

    inv = 1.0 / _B
    loss = (blk(0) + blk(1) + blk(2) + blk(3)) * inv \
        + 2.0 * (blk_sqrt(1) * inv) * (blk_sqrt(2) * inv)
    o_ref[...] = jnp.full((1, 1), loss, jnp.float32)


def kernel(class_emb, rel_emb, nf1, nf2, nf3):
    i32 = jnp.int32
    table = jnp.concatenate(
        [class_emb, jnp.pad(rel_emb, ((0, 0), (0, _D)))], axis=0)
    cols = jnp.stack(
        [nf1[:, 0].astype(i32), nf1[:, 1].astype(i32),
         nf2[:, 0].astype(i32), nf2[:, 1].astype(i32), nf2[:, 2].astype(i32),
         nf3[:, 0].astype(i32), nf3[:, 2].astype(i32),
         nf3[:, 1].astype(i32) + 1000],
        axis=0)
    nfall = cols.reshape(8, _NW, _BW).transpose(1, 0, 2).reshape(-1)
    partials = _sc_partials(table, nfall)
    out = pl.pallas_call(
        _reduce_body,
        out_shape=jax.ShapeDtypeStruct((1, 1), jnp.float32),
    )(partials.reshape(_L, 2 * 64))
    return out[0, 0]
